# Initial kernel scaffold; baseline (speedup 1.0000x reference)
import jax
import jax.numpy as jnp
from jax.experimental import pallas as pl


def kernel(index, sample_h, sample_pos):
    raise NotImplementedError("write your pallas kernel here")


# speedup vs baseline: 2.9479x; 2.9479x over previous
import dataclasses
import functools

import jax
import jax.numpy as jnp
from jax import lax
from jax.experimental import pallas as pl
from jax.experimental.pallas import tpu as pltpu
from jax.experimental.pallas import tpu_sc as plsc

N = 160000
NUM_SEGMENTS = 10000
SEGP = 10240
SEG_SLICE = SEGP // 16
CHUNK = 128
NCHUNK = N // CHUNK
NC, NS = 2, 16
NW = NC * NS
CPW = -(-NCHUNK // NW)
L = 16

_mesh = plsc.VectorSubcoreMesh(core_axis_name="c", subcore_axis_name="s")

_cp = pltpu.CompilerParams()
if "needs_layout_passes" in pltpu.CompilerParams.__dataclass_fields__:
    _cp = dataclasses.replace(_cp, needs_layout_passes=False)


def _worker_id():
    return lax.axis_index("s") * NC + lax.axis_index("c")


def _accumulate_body(index_hbm, pos_hbm, part_hbm,
                     idxb, posb, xb, yb, zb, ones, zeros,
                     accx, accy, accz, accc, sem):
    cid = lax.axis_index("c")
    sid = lax.axis_index("s")
    w = _worker_id()

    one16 = jnp.full((L,), 1.0, jnp.float32)
    zero16 = jnp.zeros((L,), jnp.float32)
    for k in range(CHUNK // L):
        ones[pl.ds(k * L, L)] = one16
    for k in range(SEG_SLICE // L):
        zeros[pl.ds(k * L, L)] = zero16

    off = sid * SEG_SLICE
    pltpu.sync_copy(zeros, accx.at[pl.ds(off, SEG_SLICE)])
    pltpu.sync_copy(zeros, accy.at[pl.ds(off, SEG_SLICE)])
    pltpu.sync_copy(zeros, accz.at[pl.ds(off, SEG_SLICE)])
    pltpu.sync_copy(zeros, accc.at[pl.ds(off, SEG_SLICE)])
    plsc.subcore_barrier()

    iota3 = lax.iota(jnp.int32, L) * 3

    @pl.loop(0, CPW)
    def _(j):
        c = w + j * NW

        @pl.when(c < NCHUNK)
        def _():
            pltpu.sync_copy(index_hbm.at[pl.ds(c, 1)], idxb)
            pltpu.sync_copy(pos_hbm.at[pl.ds(c * (3 * CHUNK), 3 * CHUNK)], posb)
            for k in range(CHUNK // L):
                base = k * 3 * L
                xb[pl.ds(k * L, L)] = plsc.load_gather(posb, [iota3 + base])
                yb[pl.ds(k * L, L)] = plsc.load_gather(posb, [iota3 + (base + 1)])
                zb[pl.ds(k * L, L)] = plsc.load_gather(posb, [iota3 + (base + 2)])
            idx = idxb.at[0]
            d1 = pltpu.async_copy(xb, accx.at[idx], sem, add=True)
            d2 = pltpu.async_copy(yb, accy.at[idx], sem, add=True)
            d3 = pltpu.async_copy(zb, accz.at[idx], sem, add=True)
            d4 = pltpu.async_copy(ones, accc.at[idx], sem, add=True)
            d1.wait()
            d2.wait()
            d3.wait()
            d4.wait()

    plsc.subcore_barrier()
    for col, acc in ((0, accx), (1, accy), (2, accz), (3, accc)):
        pltpu.sync_copy(acc.at[pl.ds(off, SEG_SLICE)],
                        part_hbm.at[cid, col, pl.ds(off, SEG_SLICE)])


def _apply_body(part_hbm, index_hbm, pos_hbm, out_hbm,
                idxb, posb, outb, pa, pb, invb, tmpb, mxb, myb, mzb,
                mx, my, mz, sem):
    sid = lax.axis_index("s")
    w = _worker_id()

    off = sid * SEG_SLICE
    pltpu.sync_copy(part_hbm.at[0, 3, pl.ds(off, SEG_SLICE)], pa)
    pltpu.sync_copy(part_hbm.at[1, 3, pl.ds(off, SEG_SLICE)], pb)
    one16 = jnp.full((L,), 1.0, jnp.float32)
    for k in range(SEG_SLICE // L):
        s = pl.ds(k * L, L)
        cnt = pa[s] + pb[s]
        invb[s] = one16 / jnp.maximum(cnt, one16)
    for col, m in ((0, mx), (1, my), (2, mz)):
        pltpu.sync_copy(part_hbm.at[0, col, pl.ds(off, SEG_SLICE)], pa)
        pltpu.sync_copy(part_hbm.at[1, col, pl.ds(off, SEG_SLICE)], pb)
        for k in range(SEG_SLICE // L):
            s = pl.ds(k * L, L)
            tmpb[s] = (pa[s] + pb[s]) * invb[s]
        pltpu.sync_copy(tmpb, m.at[pl.ds(off, SEG_SLICE)])
    plsc.subcore_barrier()

    iota3 = lax.iota(jnp.int32, L) * 3

    @pl.loop(0, CPW)
    def _(j):
        c = w + j * NW

        @pl.when(c < NCHUNK)
        def _():
            pltpu.sync_copy(index_hbm.at[pl.ds(c, 1)], idxb)
            idx = idxb.at[0]
            g1 = pltpu.async_copy(mx.at[idx], mxb, sem)
            g2 = pltpu.async_copy(my.at[idx], myb, sem)
            g3 = pltpu.async_copy(mz.at[idx], mzb, sem)
            pltpu.sync_copy(pos_hbm.at[pl.ds(c * (3 * CHUNK), 3 * CHUNK)], posb)
            g1.wait()
            g2.wait()
            g3.wait()
            for k in range(CHUNK // L):
                base = k * 3 * L
                s = pl.ds(k * L, L)
                i0 = iota3 + base
                i1 = iota3 + (base + 1)
                i2 = iota3 + (base + 2)
                plsc.store_scatter(outb, [i0], plsc.load_gather(posb, [i0]) - mxb[s])
                plsc.store_scatter(outb, [i1], plsc.load_gather(posb, [i1]) - myb[s])
                plsc.store_scatter(outb, [i2], plsc.load_gather(posb, [i2]) - mzb[s])
            pltpu.sync_copy(outb, out_hbm.at[pl.ds(c * (3 * CHUNK), 3 * CHUNK)])


@jax.jit
def _center(index2d, pos_flat):
    f32 = jnp.float32
    part = pl.kernel(
        _accumulate_body,
        out_type=jax.ShapeDtypeStruct((NC, 4, SEGP), f32),
        mesh=_mesh,
        compiler_params=_cp,
        scratch_types=[
            pltpu.VMEM((1, CHUNK), jnp.int32),
            pltpu.VMEM((3 * CHUNK,), f32),
            pltpu.VMEM((CHUNK,), f32),
            pltpu.VMEM((CHUNK,), f32),
            pltpu.VMEM((CHUNK,), f32),
            pltpu.VMEM((CHUNK,), f32),
            pltpu.VMEM((SEG_SLICE,), f32),
            pltpu.VMEM_SHARED((SEGP,), f32),
            pltpu.VMEM_SHARED((SEGP,), f32),
            pltpu.VMEM_SHARED((SEGP,), f32),
            pltpu.VMEM_SHARED((SEGP,), f32),
            pltpu.SemaphoreType.DMA,
        ],
    )(index2d, pos_flat)

    out_flat = pl.kernel(
        _apply_body,
        out_type=jax.ShapeDtypeStruct((3 * N,), f32),
        mesh=_mesh,
        compiler_params=_cp,
        scratch_types=[
            pltpu.VMEM((1, CHUNK), jnp.int32),
            pltpu.VMEM((3 * CHUNK,), f32),
            pltpu.VMEM((3 * CHUNK,), f32),
            pltpu.VMEM((SEG_SLICE,), f32),
            pltpu.VMEM((SEG_SLICE,), f32),
            pltpu.VMEM((SEG_SLICE,), f32),
            pltpu.VMEM((SEG_SLICE,), f32),
            pltpu.VMEM((CHUNK,), f32),
            pltpu.VMEM((CHUNK,), f32),
            pltpu.VMEM((CHUNK,), f32),
            pltpu.VMEM_SHARED((SEGP,), f32),
            pltpu.VMEM_SHARED((SEGP,), f32),
            pltpu.VMEM_SHARED((SEGP,), f32),
            pltpu.SemaphoreType.DMA,
        ],
    )(part, index2d, pos_flat)
    return out_flat


def kernel(index, sample_h, sample_pos):
    index2d = index.astype(jnp.int32).reshape(NCHUNK, CHUNK)
    pos_flat = sample_pos.reshape(-1)
    out_flat = _center(index2d, pos_flat)
    return (sample_h, out_flat.reshape(N, 3))
